# baseline (speedup 1.0000x reference)
import functools

import jax
import jax.numpy as jnp
from jax import lax
from jax.experimental import pallas as pl
from jax.experimental.pallas import tpu as pltpu
from jax.experimental.pallas import tpu_sc as plsc

_L = 16
_NC = 2
_NS = 16
_NW = _NC * _NS
_NBUF = 4


def _project(table, W1, W2):
    V, E = table.shape
    N1 = W1.shape[1]
    BLK = 8000

    def body(t_ref, w1_ref, w2_ref, o_ref):
        wc = jnp.dot(w1_ref[...], w2_ref[...],
                     preferred_element_type=jnp.float32,
                     precision=lax.Precision.HIGHEST)
        wc16 = jnp.concatenate(
            [wc, jnp.zeros((E, _L - wc.shape[1]), jnp.float32)], axis=1)
        o_ref[...] = jnp.dot(t_ref[...].astype(jnp.bfloat16),
                             wc16.astype(jnp.bfloat16),
                             preferred_element_type=jnp.float32)

    return pl.pallas_call(
        body,
        grid=(V // BLK,),
        in_specs=[
            pl.BlockSpec((BLK, E), lambda i: (i, 0)),
            pl.BlockSpec((E, N1), lambda i: (0, 0)),
            pl.BlockSpec((N1, W2.shape[1]), lambda i: (0, 0)),
        ],
        out_specs=pl.BlockSpec((BLK, _L), lambda i: (i, 0)),
        out_shape=jax.ShapeDtypeStruct((V, _L), jnp.float32),
    )(table, W1, W2)


def _sc_pool(proj, ids_w, B, S):
    b_per_w = B // _NW
    mesh = plsc.VectorSubcoreMesh(core_axis_name="c", subcore_axis_name="s")

    @functools.partial(
        pl.kernel,
        out_type=jax.ShapeDtypeStruct((B, _L), jnp.float32),
        mesh=mesh,
        compiler_params=pltpu.CompilerParams(use_tc_tiling_on_sc=False),
        scratch_types=(
            [pltpu.VMEM((S, b_per_w), jnp.int32),
             pltpu.VMEM((b_per_w, _L), jnp.float32)]
            + [pltpu.VMEM((b_per_w, _L), jnp.float32) for _ in range(_NBUF)]
            + [pltpu.SemaphoreType.DMA for _ in range(_NBUF)]
        ),
    )
    def k(proj_hbm, ids_hbm, out_hbm, idx_v, stage_v, b0, b1, b2, b3,
          s0, s1, s2, s3):
        bufs = (b0, b1, b2, b3)
        sems = (s0, s1, s2, s3)
        wid = lax.axis_index("s") * _NC + lax.axis_index("c")
        base = wid * b_per_w
        pltpu.sync_copy(ids_hbm.at[wid], idx_v)

        def issue(s, buf, sem):
            pltpu.async_copy(proj_hbm.at[idx_v.at[s]], buf, sem)

        def wait(buf, sem):
            pltpu.make_async_copy(proj_hbm.at[idx_v.at[0]], buf, sem).wait()

        def accum(buf):
            @pl.loop(0, b_per_w, step=8)
            def _(r):
                for u in range(8):
                    stage_v[r + u] = stage_v[r + u] + buf[r + u]

        @pl.loop(0, b_per_w)
        def _(r):
            stage_v[r] = jnp.zeros((_L,), jnp.float32)

        for b in range(_NBUF):
            issue(b, bufs[b], sems[b])

        @pl.loop(0, S, step=_NBUF)
        def _(s):
            for b in range(_NBUF):
                wait(bufs[b], sems[b])
                accum(bufs[b])

                @pl.when(s + _NBUF + b < S)
                def _():
                    issue(s + _NBUF + b, bufs[b], sems[b])

        pltpu.sync_copy(stage_v, out_hbm.at[pl.ds(base, b_per_w)])

    return k(proj, ids_w)


def _finish(sums, b1, W2, b2, S):
    B = sums.shape[0]

    def body(s_ref, b1_ref, w2_ref, b2_ref, o_ref):
        bc = jnp.dot(b1_ref[...], w2_ref[...],
                     preferred_element_type=jnp.float32,
                     precision=lax.Precision.HIGHEST) + b2_ref[...]
        z = s_ref[...][:, :2] * (1.0 / S) + bc
        m = jnp.max(z, axis=1, keepdims=True)
        e = jnp.exp(z - m)
        o_ref[...] = e / jnp.sum(e, axis=1, keepdims=True)

    return pl.pallas_call(
        body,
        out_shape=jax.ShapeDtypeStruct((B, 2), jnp.float32),
    )(sums, b1.reshape(1, -1), W2, b2.reshape(1, -1))


def kernel(input_ids, table, W1, b1, W2, b2):
    B, S = input_ids.shape
    b_per_w = B // _NW
    proj = _project(table, W1, W2)
    ids_w = jnp.transpose(
        input_ids.astype(jnp.int32).reshape(_NW, b_per_w, S), (0, 2, 1))
    sums = _sc_pool(proj, ids_w, B, S)
    return _finish(sums, b1, W2, b2, S)


# speedup vs baseline: 1.5527x; 1.5527x over previous
import functools

import jax
import jax.numpy as jnp
from jax import lax
from jax.experimental import pallas as pl
from jax.experimental.pallas import tpu as pltpu
from jax.experimental.pallas import tpu_sc as plsc

_L = 16
_NC = 2
_NS = 16
_NW = _NC * _NS
_NBUF = 4


_SUB = 1000
_VBLK = 8 * _SUB


def _project(table, W1, W2):
    V, E = table.shape
    N1 = W1.shape[1]

    def body(t_ref, w1_ref, w2_ref, o_ref):
        wc = jnp.dot(w1_ref[...], w2_ref[...],
                     preferred_element_type=jnp.float32,
                     precision=lax.Precision.HIGHEST)
        wc16 = jnp.concatenate(
            [wc, jnp.zeros((E, _L - wc.shape[1]), jnp.float32)], axis=1)
        x = t_ref[...].astype(jnp.bfloat16)
        wcb = wc16.astype(jnp.bfloat16)
        o_ref[...] = jnp.concatenate(
            [jnp.dot(x[k * _SUB:(k + 1) * _SUB], wcb,
                     preferred_element_type=jnp.float32)
             for k in range(8)], axis=1)

    return pl.pallas_call(
        body,
        grid=(V // _VBLK,),
        in_specs=[
            pl.BlockSpec((_VBLK, E), lambda i: (i, 0)),
            pl.BlockSpec((E, N1), lambda i: (0, 0)),
            pl.BlockSpec((N1, W2.shape[1]), lambda i: (0, 0)),
        ],
        out_specs=pl.BlockSpec((_SUB, 8 * _L), lambda i: (i, 0)),
        out_shape=jax.ShapeDtypeStruct((V // 8, 8 * _L), jnp.float32),
    )(table, W1, W2)


def _permute_ids(ids):
    o = ids % _VBLK
    return ids - o + (o % _SUB) * 8 + o // _SUB


def _sc_pool(proj, ids_w, B, S):
    b_per_w = B // _NW
    mesh = plsc.VectorSubcoreMesh(core_axis_name="c", subcore_axis_name="s")

    @functools.partial(
        pl.kernel,
        out_type=jax.ShapeDtypeStruct((B, _L), jnp.float32),
        mesh=mesh,
        compiler_params=pltpu.CompilerParams(use_tc_tiling_on_sc=False),
        scratch_types=(
            [pltpu.VMEM((S, b_per_w), jnp.int32),
             pltpu.VMEM((b_per_w, _L), jnp.float32)]
            + [pltpu.VMEM((b_per_w, _L), jnp.float32) for _ in range(_NBUF)]
            + [pltpu.SemaphoreType.DMA for _ in range(_NBUF)]
        ),
    )
    def k(proj_hbm, ids_hbm, out_hbm, idx_v, stage_v, b0, b1, b2, b3,
          s0, s1, s2, s3):
        bufs = (b0, b1, b2, b3)
        sems = (s0, s1, s2, s3)
        wid = lax.axis_index("s") * _NC + lax.axis_index("c")
        base = wid * b_per_w
        pltpu.sync_copy(ids_hbm.at[wid], idx_v)

        def issue(s, buf, sem):
            pltpu.async_copy(proj_hbm.at[idx_v.at[s]], buf, sem)

        def wait(buf, sem):
            pltpu.make_async_copy(proj_hbm.at[idx_v.at[0]], buf, sem).wait()

        def accum(buf):
            @pl.loop(0, b_per_w, step=8)
            def _(r):
                for u in range(8):
                    stage_v[r + u] = stage_v[r + u] + buf[r + u]

        @pl.loop(0, b_per_w)
        def _(r):
            stage_v[r] = jnp.zeros((_L,), jnp.float32)

        for b in range(_NBUF):
            issue(b, bufs[b], sems[b])

        @pl.loop(0, S, step=_NBUF)
        def _(s):
            for b in range(_NBUF):
                wait(bufs[b], sems[b])
                accum(bufs[b])

                @pl.when(s + _NBUF + b < S)
                def _():
                    issue(s + _NBUF + b, bufs[b], sems[b])

        pltpu.sync_copy(stage_v, out_hbm.at[pl.ds(base, b_per_w)])

    return k(proj, ids_w)


def _finish(sums, b1, W2, b2, S):
    B = sums.shape[0]

    def body(s_ref, b1_ref, w2_ref, b2_ref, o_ref):
        bc = jnp.dot(b1_ref[...], w2_ref[...],
                     preferred_element_type=jnp.float32,
                     precision=lax.Precision.HIGHEST) + b2_ref[...]
        z = s_ref[...][:, :2] * (1.0 / S) + bc
        m = jnp.max(z, axis=1, keepdims=True)
        e = jnp.exp(z - m)
        o_ref[...] = e / jnp.sum(e, axis=1, keepdims=True)

    return pl.pallas_call(
        body,
        out_shape=jax.ShapeDtypeStruct((B, 2), jnp.float32),
    )(sums, b1.reshape(1, -1), W2, b2.reshape(1, -1))


def kernel(input_ids, table, W1, b1, W2, b2):
    B, S = input_ids.shape
    b_per_w = B // _NW
    proj = _project(table, W1, W2).reshape(table.shape[0], _L)
    ids_w = jnp.transpose(
        _permute_ids(input_ids.astype(jnp.int32)).reshape(_NW, b_per_w, S),
        (0, 2, 1))
    sums = _sc_pool(proj, ids_w, B, S)
    return _finish(sums, b1, W2, b2, S)


# speedup vs baseline: 3.3628x; 2.1657x over previous
import functools

import jax
import jax.numpy as jnp
from jax import lax
from jax.experimental import pallas as pl
from jax.experimental.pallas import tpu as pltpu
from jax.experimental.pallas import tpu_sc as plsc

_L = 16
_NC = 2
_NS = 16
_NW = _NC * _NS
_NBUF = 4


_SUB = 2000
_VBLK = 8 * _SUB


def _project(table, W1, W2):
    V, E = table.shape
    N1 = W1.shape[1]
    tt = table.T

    def body(t_ref, w1_ref, w2_ref, o_ref):
        wc = jnp.dot(w1_ref[...], w2_ref[...],
                     preferred_element_type=jnp.float32,
                     precision=lax.Precision.HIGHEST)
        wc16 = jnp.concatenate(
            [wc, jnp.zeros((E, _L - wc.shape[1]), jnp.float32)], axis=1)
        x = t_ref[...].astype(jnp.bfloat16)
        wcb = wc16.astype(jnp.bfloat16)
        o_ref[...] = jnp.concatenate(
            [lax.dot_general(x[:, k * _SUB:(k + 1) * _SUB], wcb,
                             (((0,), (0,)), ((), ())),
                             preferred_element_type=jnp.float32)
             for k in range(8)], axis=1)

    return pl.pallas_call(
        body,
        grid=(V // _VBLK,),
        in_specs=[
            pl.BlockSpec((E, _VBLK), lambda i: (0, i)),
            pl.BlockSpec((E, N1), lambda i: (0, 0)),
            pl.BlockSpec((N1, W2.shape[1]), lambda i: (0, 0)),
        ],
        out_specs=pl.BlockSpec((_SUB, 8 * _L), lambda i: (i, 0)),
        out_shape=jax.ShapeDtypeStruct((V // 8, 8 * _L), jnp.float32),
        compiler_params=pltpu.CompilerParams(
            dimension_semantics=("parallel",)),
    )(tt, W1, W2)


def _permute_ids(ids):
    o = ids % _VBLK
    return ids - o + (o % _SUB) * 8 + o // _SUB


def _sc_pool(proj, ids_w, B, S):
    b_per_w = B // _NW
    mesh = plsc.VectorSubcoreMesh(core_axis_name="c", subcore_axis_name="s")

    @functools.partial(
        pl.kernel,
        out_type=jax.ShapeDtypeStruct((B, _L), jnp.float32),
        mesh=mesh,
        compiler_params=pltpu.CompilerParams(use_tc_tiling_on_sc=False),
        scratch_types=(
            [pltpu.VMEM((S, b_per_w), jnp.int32),
             pltpu.VMEM((b_per_w, _L), jnp.float32)]
            + [pltpu.VMEM((b_per_w, _L), jnp.float32) for _ in range(_NBUF)]
            + [pltpu.SemaphoreType.DMA for _ in range(_NBUF)]
        ),
    )
    def k(proj_hbm, ids_hbm, out_hbm, idx_v, stage_v, b0, b1, b2, b3,
          s0, s1, s2, s3):
        bufs = (b0, b1, b2, b3)
        sems = (s0, s1, s2, s3)
        wid = lax.axis_index("s") * _NC + lax.axis_index("c")
        base = wid * b_per_w
        pltpu.sync_copy(ids_hbm.at[wid], idx_v)

        def issue(s, buf, sem):
            pltpu.async_copy(proj_hbm.at[idx_v.at[s]], buf, sem)

        def wait(buf, sem):
            pltpu.make_async_copy(proj_hbm.at[idx_v.at[0]], buf, sem).wait()

        def accum(buf):
            @pl.loop(0, b_per_w, step=8)
            def _(r):
                for u in range(8):
                    stage_v[r + u] = stage_v[r + u] + buf[r + u]

        @pl.loop(0, b_per_w)
        def _(r):
            stage_v[r] = jnp.zeros((_L,), jnp.float32)

        for b in range(_NBUF):
            issue(b, bufs[b], sems[b])

        @pl.loop(0, S, step=_NBUF)
        def _(s):
            for b in range(_NBUF):
                wait(bufs[b], sems[b])
                accum(bufs[b])

                @pl.when(s + _NBUF + b < S)
                def _():
                    issue(s + _NBUF + b, bufs[b], sems[b])

        pltpu.sync_copy(stage_v, out_hbm.at[pl.ds(base, b_per_w)])

    return k(proj, ids_w)


def _finish(sums, b1, W2, b2, S):
    B = sums.shape[0]

    def body(s_ref, b1_ref, w2_ref, b2_ref, o_ref):
        bc = jnp.dot(b1_ref[...], w2_ref[...],
                     preferred_element_type=jnp.float32,
                     precision=lax.Precision.HIGHEST) + b2_ref[...]
        z = s_ref[...][:, :2] * (1.0 / S) + bc
        m = jnp.max(z, axis=1, keepdims=True)
        e = jnp.exp(z - m)
        o_ref[...] = e / jnp.sum(e, axis=1, keepdims=True)

    return pl.pallas_call(
        body,
        out_shape=jax.ShapeDtypeStruct((B, 2), jnp.float32),
    )(sums, b1.reshape(1, -1), W2, b2.reshape(1, -1))


def kernel(input_ids, table, W1, b1, W2, b2):
    B, S = input_ids.shape
    b_per_w = B // _NW
    proj = _project(table, W1, W2).reshape(table.shape[0], _L)
    ids_w = jnp.transpose(
        _permute_ids(input_ids.astype(jnp.int32)).reshape(_NW, b_per_w, S),
        (0, 2, 1))
    sums = _sc_pool(proj, ids_w, B, S)
    return _finish(sums, b1, W2, b2, S)


# speedup vs baseline: 3.3705x; 1.0023x over previous
import functools

import jax
import jax.numpy as jnp
from jax import lax
from jax.experimental import pallas as pl
from jax.experimental.pallas import tpu as pltpu
from jax.experimental.pallas import tpu_sc as plsc

_L = 16
_NC = 2
_NS = 16
_NW = _NC * _NS
_NBUF = 4


_SUB = 2016
_VBLK = 8 * _SUB


def _project(table, W1, W2):
    V, E = table.shape
    N1 = W1.shape[1]
    tt = table.T

    def body(t_ref, w1_ref, w2_ref, o_ref):
        wc = jnp.dot(w1_ref[...], w2_ref[...],
                     preferred_element_type=jnp.float32,
                     precision=lax.Precision.HIGHEST)
        wc16 = jnp.concatenate(
            [wc, jnp.zeros((E, _L - wc.shape[1]), jnp.float32)], axis=1)
        x = t_ref[...].astype(jnp.bfloat16)
        wcb = wc16.astype(jnp.bfloat16)
        o_ref[...] = jnp.concatenate(
            [lax.dot_general(x[:, k * _SUB:(k + 1) * _SUB], wcb,
                             (((0,), (0,)), ((), ())),
                             preferred_element_type=jnp.float32)
             for k in range(8)], axis=1)

    g = -(-V // _VBLK)
    return pl.pallas_call(
        body,
        grid=(g,),
        in_specs=[
            pl.BlockSpec((E, _VBLK), lambda i: (0, i)),
            pl.BlockSpec((E, N1), lambda i: (0, 0)),
            pl.BlockSpec((N1, W2.shape[1]), lambda i: (0, 0)),
        ],
        out_specs=pl.BlockSpec((_SUB, 8 * _L), lambda i: (i, 0)),
        out_shape=jax.ShapeDtypeStruct((g * _SUB, 8 * _L), jnp.float32),
        compiler_params=pltpu.CompilerParams(
            dimension_semantics=("parallel",),
            fuse_transposed_lhs_in_matmul=True),
    )(tt, W1, W2)


def _permute_ids(ids):
    o = ids % _VBLK
    return ids - o + (o % _SUB) * 8 + o // _SUB


def _sc_pool(proj, ids_w, B, S):
    b_per_w = B // _NW
    mesh = plsc.VectorSubcoreMesh(core_axis_name="c", subcore_axis_name="s")

    @functools.partial(
        pl.kernel,
        out_type=jax.ShapeDtypeStruct((B, _L), jnp.float32),
        mesh=mesh,
        compiler_params=pltpu.CompilerParams(use_tc_tiling_on_sc=False),
        scratch_types=(
            [pltpu.VMEM((S, b_per_w), jnp.int32),
             pltpu.VMEM((b_per_w, _L), jnp.float32)]
            + [pltpu.VMEM((b_per_w, _L), jnp.float32) for _ in range(_NBUF)]
            + [pltpu.SemaphoreType.DMA for _ in range(_NBUF)]
        ),
    )
    def k(proj_hbm, ids_hbm, out_hbm, idx_v, stage_v, b0, b1, b2, b3,
          s0, s1, s2, s3):
        bufs = (b0, b1, b2, b3)
        sems = (s0, s1, s2, s3)
        wid = lax.axis_index("s") * _NC + lax.axis_index("c")
        base = wid * b_per_w
        pltpu.sync_copy(ids_hbm.at[wid], idx_v)

        def issue(s, buf, sem):
            pltpu.async_copy(proj_hbm.at[idx_v.at[s]], buf, sem)

        def wait(buf, sem):
            pltpu.make_async_copy(proj_hbm.at[idx_v.at[0]], buf, sem).wait()

        def accum(buf):
            @pl.loop(0, b_per_w, step=8)
            def _(r):
                for u in range(8):
                    stage_v[r + u] = stage_v[r + u] + buf[r + u]

        @pl.loop(0, b_per_w)
        def _(r):
            stage_v[r] = jnp.zeros((_L,), jnp.float32)

        for b in range(_NBUF):
            issue(b, bufs[b], sems[b])

        @pl.loop(0, S, step=_NBUF)
        def _(s):
            for b in range(_NBUF):
                wait(bufs[b], sems[b])
                accum(bufs[b])

                @pl.when(s + _NBUF + b < S)
                def _():
                    issue(s + _NBUF + b, bufs[b], sems[b])

        pltpu.sync_copy(stage_v, out_hbm.at[pl.ds(base, b_per_w)])

    return k(proj, ids_w)


def _finish(sums, b1, W2, b2, S):
    B = sums.shape[0]

    def body(s_ref, b1_ref, w2_ref, b2_ref, o_ref):
        bc = jnp.dot(b1_ref[...], w2_ref[...],
                     preferred_element_type=jnp.float32,
                     precision=lax.Precision.HIGHEST) + b2_ref[...]
        z = s_ref[...][:, :2] * (1.0 / S) + bc
        m = jnp.max(z, axis=1, keepdims=True)
        e = jnp.exp(z - m)
        o_ref[...] = e / jnp.sum(e, axis=1, keepdims=True)

    return pl.pallas_call(
        body,
        out_shape=jax.ShapeDtypeStruct((B, 2), jnp.float32),
    )(sums, b1.reshape(1, -1), W2, b2.reshape(1, -1))


def kernel(input_ids, table, W1, b1, W2, b2):
    B, S = input_ids.shape
    b_per_w = B // _NW
    proj = _project(table, W1, W2).reshape(-1, _L)
    ids_w = jnp.transpose(
        _permute_ids(input_ids.astype(jnp.int32)).reshape(_NW, b_per_w, S),
        (0, 2, 1))
    sums = _sc_pool(proj, ids_w, B, S)
    return _finish(sums, b1, W2, b2, S)


# speedup vs baseline: 4.3650x; 1.2951x over previous
import functools

import jax
import jax.numpy as jnp
from jax import lax
from jax.experimental import pallas as pl
from jax.experimental.pallas import tpu as pltpu
from jax.experimental.pallas import tpu_sc as plsc

_L = 16
_NC = 2
_NS = 16
_NW = _NC * _NS
_NBUF = 4


_SUB = 2016
_VBLK = 8 * _SUB


def _project(table, W1, W2):
    V, E = table.shape
    N1 = W1.shape[1]
    tt = table.T

    def body(t_ref, w1_ref, w2_ref, o_ref):
        wc = jnp.dot(w1_ref[...], w2_ref[...],
                     preferred_element_type=jnp.float32,
                     precision=lax.Precision.HIGHEST)
        wc16 = jnp.concatenate(
            [wc, jnp.zeros((E, _L - wc.shape[1]), jnp.float32)], axis=1)
        nvalid = V - pl.program_id(0) * _VBLK
        col = lax.broadcasted_iota(jnp.int32, (E, _VBLK), 1)
        x = jnp.where(col < nvalid, t_ref[...], 0.0).astype(jnp.bfloat16)
        wcb = wc16.astype(jnp.bfloat16)
        acc = None
        for k in range(8):
            wk = jnp.pad(wcb, ((0, 0), (16 * k, 112 - 16 * k)))
            yk = lax.dot_general(x[:, k * _SUB:(k + 1) * _SUB], wk,
                                 (((0,), (0,)), ((), ())),
                                 preferred_element_type=jnp.float32)
            acc = yk if acc is None else acc + yk
        o_ref[...] = acc

    g = -(-V // _VBLK)
    return pl.pallas_call(
        body,
        grid=(g,),
        in_specs=[
            pl.BlockSpec((E, _VBLK), lambda i: (0, i)),
            pl.BlockSpec((E, N1), lambda i: (0, 0)),
            pl.BlockSpec((N1, W2.shape[1]), lambda i: (0, 0)),
        ],
        out_specs=pl.BlockSpec((_SUB, 8 * _L), lambda i: (i, 0)),
        out_shape=jax.ShapeDtypeStruct((g * _SUB, 8 * _L), jnp.float32),
        compiler_params=pltpu.CompilerParams(
            dimension_semantics=("parallel",),
            fuse_transposed_lhs_in_matmul=True),
    )(tt, W1, W2)


def _permute_ids(ids):
    o = ids % _VBLK
    return ids - o + (o % _SUB) * 8 + o // _SUB


def _sc_pool(proj, ids_w, B, S):
    b_per_w = B // _NW
    mesh = plsc.VectorSubcoreMesh(core_axis_name="c", subcore_axis_name="s")

    @functools.partial(
        pl.kernel,
        out_type=jax.ShapeDtypeStruct((B, _L), jnp.float32),
        mesh=mesh,
        compiler_params=pltpu.CompilerParams(use_tc_tiling_on_sc=False),
        scratch_types=(
            [pltpu.VMEM((S, b_per_w), jnp.int32),
             pltpu.VMEM((b_per_w, _L), jnp.float32)]
            + [pltpu.VMEM((b_per_w, _L), jnp.float32) for _ in range(_NBUF)]
            + [pltpu.SemaphoreType.DMA for _ in range(_NBUF)]
        ),
    )
    def k(proj_hbm, ids_hbm, out_hbm, idx_v, stage_v, b0, b1, b2, b3,
          s0, s1, s2, s3):
        bufs = (b0, b1, b2, b3)
        sems = (s0, s1, s2, s3)
        wid = lax.axis_index("s") * _NC + lax.axis_index("c")
        base = wid * b_per_w
        pltpu.sync_copy(ids_hbm.at[wid], idx_v)

        def issue(s, buf, sem):
            pltpu.async_copy(proj_hbm.at[idx_v.at[s]], buf, sem)

        def wait(buf, sem):
            pltpu.make_async_copy(proj_hbm.at[idx_v.at[0]], buf, sem).wait()

        def accum(buf):
            @pl.loop(0, b_per_w, step=8)
            def _(r):
                for u in range(8):
                    stage_v[r + u] = stage_v[r + u] + buf[r + u]

        @pl.loop(0, b_per_w)
        def _(r):
            stage_v[r] = jnp.zeros((_L,), jnp.float32)

        for b in range(_NBUF):
            issue(b, bufs[b], sems[b])

        @pl.loop(0, S, step=_NBUF)
        def _(s):
            for b in range(_NBUF):
                wait(bufs[b], sems[b])
                accum(bufs[b])

                @pl.when(s + _NBUF + b < S)
                def _():
                    issue(s + _NBUF + b, bufs[b], sems[b])

        pltpu.sync_copy(stage_v, out_hbm.at[pl.ds(base, b_per_w)])

    return k(proj, ids_w)


def _finish(sums, b1, W2, b2, S):
    B = sums.shape[0]

    def body(s_ref, b1_ref, w2_ref, b2_ref, o_ref):
        bc = jnp.dot(b1_ref[...], w2_ref[...],
                     preferred_element_type=jnp.float32,
                     precision=lax.Precision.HIGHEST) + b2_ref[...]
        z = s_ref[...][:, :2] * (1.0 / S) + bc
        m = jnp.max(z, axis=1, keepdims=True)
        e = jnp.exp(z - m)
        o_ref[...] = e / jnp.sum(e, axis=1, keepdims=True)

    return pl.pallas_call(
        body,
        out_shape=jax.ShapeDtypeStruct((B, 2), jnp.float32),
    )(sums, b1.reshape(1, -1), W2, b2.reshape(1, -1))


def kernel(input_ids, table, W1, b1, W2, b2):
    B, S = input_ids.shape
    b_per_w = B // _NW
    proj = _project(table, W1, W2).reshape(-1, _L)
    ids_w = jnp.transpose(
        _permute_ids(input_ids.astype(jnp.int32)).reshape(_NW, b_per_w, S),
        (0, 2, 1))
    sums = _sc_pool(proj, ids_w, B, S)
    return _finish(sums, b1, W2, b2, S)


# speedup vs baseline: 4.9265x; 1.1286x over previous
import functools

import jax
import jax.numpy as jnp
from jax import lax
from jax.experimental import pallas as pl
from jax.experimental.pallas import tpu as pltpu
from jax.experimental.pallas import tpu_sc as plsc

_L = 16
_NC = 2
_NS = 16
_NW = _NC * _NS
_NBUF = 4


_SUB = 4032
_VBLK = 8 * _SUB


def _project(table, W1, W2):
    V, E = table.shape
    N1 = W1.shape[1]
    tt = table.T

    g = -(-V // _VBLK)

    def body(t_ref, w1_ref, w2_ref, o_ref):
        wc = jnp.dot(w1_ref[...], w2_ref[...],
                     preferred_element_type=jnp.float32,
                     precision=lax.Precision.HIGHEST)
        wc16 = jnp.concatenate(
            [wc, jnp.zeros((E, _L - wc.shape[1]), jnp.float32)], axis=1)
        nvalid = V - pl.program_id(0) * _VBLK
        col = lax.broadcasted_iota(jnp.int32, (E, _VBLK), 1)
        x = jnp.where(col < nvalid, t_ref[...], 0.0).astype(jnp.bfloat16)
        wcb = wc16.astype(jnp.bfloat16)
        acc = None
        for k in range(8):
            wk = jnp.pad(wcb, ((0, 0), (16 * k, 112 - 16 * k)))
            yk = lax.dot_general(x[:, k * _SUB:(k + 1) * _SUB], wk,
                                 (((0,), (0,)), ((), ())),
                                 preferred_element_type=jnp.float32)
            acc = yk if acc is None else acc + yk
        o_ref[...] = acc

    return pl.pallas_call(
        body,
        grid=(g,),
        in_specs=[
            pl.BlockSpec((E, _VBLK), lambda i: (0, i)),
            pl.BlockSpec((E, N1), lambda i: (0, 0)),
            pl.BlockSpec((N1, W2.shape[1]), lambda i: (0, 0)),
        ],
        out_specs=pl.BlockSpec((_SUB, 8 * _L), lambda i: (i, 0)),
        out_shape=jax.ShapeDtypeStruct((g * _SUB, 8 * _L), jnp.float32),
        compiler_params=pltpu.CompilerParams(
            dimension_semantics=("parallel",),
            fuse_transposed_lhs_in_matmul=True),
    )(tt, W1, W2)


def _permute_ids(ids):
    o = ids % _VBLK
    return ids - o + (o % _SUB) * 8 + o // _SUB


def _sc_pool(proj, ids_w, B, S):
    b_per_w = B // _NW
    mesh = plsc.VectorSubcoreMesh(core_axis_name="c", subcore_axis_name="s")

    @functools.partial(
        pl.kernel,
        out_type=jax.ShapeDtypeStruct((B, _L), jnp.float32),
        mesh=mesh,
        compiler_params=pltpu.CompilerParams(use_tc_tiling_on_sc=False),
        scratch_types=(
            [pltpu.VMEM((S, b_per_w), jnp.int32),
             pltpu.VMEM((b_per_w, _L), jnp.float32)]
            + [pltpu.VMEM((b_per_w, _L), jnp.float32) for _ in range(_NBUF)]
            + [pltpu.SemaphoreType.DMA for _ in range(_NBUF)]
        ),
    )
    def k(proj_hbm, ids_hbm, out_hbm, idx_v, stage_v, b0, b1, b2, b3,
          s0, s1, s2, s3):
        bufs = (b0, b1, b2, b3)
        sems = (s0, s1, s2, s3)
        wid = lax.axis_index("s") * _NC + lax.axis_index("c")
        base = wid * b_per_w
        pltpu.sync_copy(ids_hbm.at[wid], idx_v)

        def issue(s, buf, sem):
            pltpu.async_copy(proj_hbm.at[idx_v.at[s]], buf, sem)

        def wait(buf, sem):
            pltpu.make_async_copy(proj_hbm.at[idx_v.at[0]], buf, sem).wait()

        def accum(buf):
            @pl.loop(0, b_per_w, step=8)
            def _(r):
                for u in range(8):
                    stage_v[r + u] = stage_v[r + u] + buf[r + u]

        @pl.loop(0, b_per_w)
        def _(r):
            stage_v[r] = jnp.zeros((_L,), jnp.float32)

        for b in range(_NBUF):
            issue(b, bufs[b], sems[b])

        @pl.loop(0, S, step=_NBUF)
        def _(s):
            for b in range(_NBUF):
                wait(bufs[b], sems[b])
                accum(bufs[b])

                @pl.when(s + _NBUF + b < S)
                def _():
                    issue(s + _NBUF + b, bufs[b], sems[b])

        pltpu.sync_copy(stage_v, out_hbm.at[pl.ds(base, b_per_w)])

    return k(proj, ids_w)


def _finish(sums, b1, W2, b2, S):
    B = sums.shape[0]

    def body(s_ref, b1_ref, w2_ref, b2_ref, o_ref):
        bc = jnp.dot(b1_ref[...], w2_ref[...],
                     preferred_element_type=jnp.float32,
                     precision=lax.Precision.HIGHEST) + b2_ref[...]
        z = s_ref[...][:, :2] * (1.0 / S) + bc
        m = jnp.max(z, axis=1, keepdims=True)
        e = jnp.exp(z - m)
        o_ref[...] = e / jnp.sum(e, axis=1, keepdims=True)

    return pl.pallas_call(
        body,
        out_shape=jax.ShapeDtypeStruct((B, 2), jnp.float32),
    )(sums, b1.reshape(1, -1), W2, b2.reshape(1, -1))


def kernel(input_ids, table, W1, b1, W2, b2):
    B, S = input_ids.shape
    b_per_w = B // _NW
    proj = _project(table, W1, W2).reshape(-1, _L)
    ids_w = jnp.transpose(
        _permute_ids(input_ids.astype(jnp.int32)).reshape(_NW, b_per_w, S),
        (0, 2, 1))
    sums = _sc_pool(proj, ids_w, B, S)
    return _finish(sums, b1, W2, b2, S)


# speedup vs baseline: 5.0207x; 1.0191x over previous
import functools

import jax
import jax.numpy as jnp
from jax import lax
from jax.experimental import pallas as pl
from jax.experimental.pallas import tpu as pltpu
from jax.experimental.pallas import tpu_sc as plsc

_L = 16
_NC = 2
_NS = 16
_NW = _NC * _NS
_NBUF = 4


_SUB = 6048
_VBLK = 8 * _SUB


def _project(table, W1, W2):
    V, E = table.shape
    N1 = W1.shape[1]
    tt = table.T

    g = -(-V // _VBLK)

    def body(t_ref, w1_ref, w2_ref, o_ref):
        wc = jnp.dot(w1_ref[...], w2_ref[...],
                     preferred_element_type=jnp.float32,
                     precision=lax.Precision.HIGHEST)
        wc16 = jnp.concatenate(
            [wc, jnp.zeros((E, _L - wc.shape[1]), jnp.float32)], axis=1)
        nvalid = V - pl.program_id(0) * _VBLK
        col = lax.broadcasted_iota(jnp.int32, (E, _VBLK), 1)
        x = jnp.where(col < nvalid, t_ref[...], 0.0).astype(jnp.bfloat16)
        wcb = wc16.astype(jnp.bfloat16)
        acc = None
        for k in range(8):
            wk = jnp.pad(wcb, ((0, 0), (16 * k, 112 - 16 * k)))
            yk = lax.dot_general(x[:, k * _SUB:(k + 1) * _SUB], wk,
                                 (((0,), (0,)), ((), ())),
                                 preferred_element_type=jnp.float32)
            acc = yk if acc is None else acc + yk
        o_ref[...] = acc

    return pl.pallas_call(
        body,
        grid=(g,),
        in_specs=[
            pl.BlockSpec((E, _VBLK), lambda i: (0, i)),
            pl.BlockSpec((E, N1), lambda i: (0, 0)),
            pl.BlockSpec((N1, W2.shape[1]), lambda i: (0, 0)),
        ],
        out_specs=pl.BlockSpec((_SUB, 8 * _L), lambda i: (i, 0)),
        out_shape=jax.ShapeDtypeStruct((g * _SUB, 8 * _L), jnp.float32),
        compiler_params=pltpu.CompilerParams(
            dimension_semantics=("parallel",),
            fuse_transposed_lhs_in_matmul=True),
    )(tt, W1, W2)


def _permute_ids(ids):
    o = ids % _VBLK
    return ids - o + (o % _SUB) * 8 + o // _SUB


def _sc_pool(proj, ids_w, B, S):
    b_per_w = B // _NW
    mesh = plsc.VectorSubcoreMesh(core_axis_name="c", subcore_axis_name="s")

    @functools.partial(
        pl.kernel,
        out_type=jax.ShapeDtypeStruct((B, _L), jnp.float32),
        mesh=mesh,
        compiler_params=pltpu.CompilerParams(use_tc_tiling_on_sc=False),
        scratch_types=(
            [pltpu.VMEM((S, b_per_w), jnp.int32),
             pltpu.VMEM((b_per_w, _L), jnp.float32)]
            + [pltpu.VMEM((b_per_w, _L), jnp.float32) for _ in range(_NBUF)]
            + [pltpu.SemaphoreType.DMA for _ in range(_NBUF)]
        ),
    )
    def k(proj_hbm, ids_hbm, out_hbm, idx_v, stage_v, b0, b1, b2, b3,
          s0, s1, s2, s3):
        bufs = (b0, b1, b2, b3)
        sems = (s0, s1, s2, s3)
        wid = lax.axis_index("s") * _NC + lax.axis_index("c")
        base = wid * b_per_w
        pltpu.sync_copy(ids_hbm.at[wid], idx_v)

        def issue(s, buf, sem):
            pltpu.async_copy(proj_hbm.at[idx_v.at[s]], buf, sem)

        def wait(buf, sem):
            pltpu.make_async_copy(proj_hbm.at[idx_v.at[0]], buf, sem).wait()

        def accum(buf):
            @pl.loop(0, b_per_w, step=8)
            def _(r):
                for u in range(8):
                    stage_v[r + u] = stage_v[r + u] + buf[r + u]

        @pl.loop(0, b_per_w)
        def _(r):
            stage_v[r] = jnp.zeros((_L,), jnp.float32)

        for b in range(_NBUF):
            issue(b, bufs[b], sems[b])

        @pl.loop(0, S, step=_NBUF)
        def _(s):
            for b in range(_NBUF):
                wait(bufs[b], sems[b])
                accum(bufs[b])

                @pl.when(s + _NBUF + b < S)
                def _():
                    issue(s + _NBUF + b, bufs[b], sems[b])

        pltpu.sync_copy(stage_v, out_hbm.at[pl.ds(base, b_per_w)])

    return k(proj, ids_w)


def _finish(sums, b1, W2, b2, S):
    B = sums.shape[0]

    def body(s_ref, b1_ref, w2_ref, b2_ref, o_ref):
        bc = jnp.dot(b1_ref[...], w2_ref[...],
                     preferred_element_type=jnp.float32,
                     precision=lax.Precision.HIGHEST) + b2_ref[...]
        z = s_ref[...][:, :2] * (1.0 / S) + bc
        m = jnp.max(z, axis=1, keepdims=True)
        e = jnp.exp(z - m)
        o_ref[...] = e / jnp.sum(e, axis=1, keepdims=True)

    return pl.pallas_call(
        body,
        out_shape=jax.ShapeDtypeStruct((B, 2), jnp.float32),
    )(sums, b1.reshape(1, -1), W2, b2.reshape(1, -1))


def kernel(input_ids, table, W1, b1, W2, b2):
    B, S = input_ids.shape
    b_per_w = B // _NW
    proj = _project(table, W1, W2).reshape(-1, _L)
    ids_w = jnp.transpose(
        _permute_ids(input_ids.astype(jnp.int32)).reshape(_NW, b_per_w, S),
        (0, 2, 1))
    sums = _sc_pool(proj, ids_w, B, S)
    return _finish(sums, b1, W2, b2, S)


# speedup vs baseline: 5.3198x; 1.0596x over previous
import functools

import jax
import jax.numpy as jnp
from jax import lax
from jax.experimental import pallas as pl
from jax.experimental.pallas import tpu as pltpu
from jax.experimental.pallas import tpu_sc as plsc

_L = 16
_NC = 2
_NS = 16
_NW = _NC * _NS
_NBUF = 8


_SUB = 6048
_VBLK = 8 * _SUB


def _project(table, W1, W2):
    V, E = table.shape
    N1 = W1.shape[1]
    tt = table.T

    g = -(-V // _VBLK)

    def body(t_ref, w1_ref, w2_ref, o_ref):
        wc = jnp.dot(w1_ref[...], w2_ref[...],
                     preferred_element_type=jnp.float32,
                     precision=lax.Precision.HIGHEST)
        wc16 = jnp.concatenate(
            [wc, jnp.zeros((E, _L - wc.shape[1]), jnp.float32)], axis=1)
        nvalid = V - pl.program_id(0) * _VBLK
        col = lax.broadcasted_iota(jnp.int32, (E, _VBLK), 1)
        x = jnp.where(col < nvalid, t_ref[...], 0.0).astype(jnp.bfloat16)
        wcb = wc16.astype(jnp.bfloat16)
        acc = None
        for k in range(8):
            wk = jnp.pad(wcb, ((0, 0), (16 * k, 112 - 16 * k)))
            yk = lax.dot_general(x[:, k * _SUB:(k + 1) * _SUB], wk,
                                 (((0,), (0,)), ((), ())),
                                 preferred_element_type=jnp.float32)
            acc = yk if acc is None else acc + yk
        o_ref[...] = acc

    return pl.pallas_call(
        body,
        grid=(g,),
        in_specs=[
            pl.BlockSpec((E, _VBLK), lambda i: (0, i)),
            pl.BlockSpec((E, N1), lambda i: (0, 0)),
            pl.BlockSpec((N1, W2.shape[1]), lambda i: (0, 0)),
        ],
        out_specs=pl.BlockSpec((_SUB, 8 * _L), lambda i: (i, 0)),
        out_shape=jax.ShapeDtypeStruct((g * _SUB, 8 * _L), jnp.float32),
        compiler_params=pltpu.CompilerParams(
            dimension_semantics=("parallel",),
            fuse_transposed_lhs_in_matmul=True),
    )(tt, W1, W2)


def _permute_ids(ids):
    o = ids % _VBLK
    return ids - o + (o % _SUB) * 8 + o // _SUB


def _sc_pool(proj, ids_w, B, S):
    b_per_w = B // _NW
    mesh = plsc.VectorSubcoreMesh(core_axis_name="c", subcore_axis_name="s")

    @functools.partial(
        pl.kernel,
        out_type=jax.ShapeDtypeStruct((B, _L), jnp.float32),
        mesh=mesh,
        compiler_params=pltpu.CompilerParams(use_tc_tiling_on_sc=False),
        scratch_types=(
            [pltpu.VMEM((S, b_per_w), jnp.int32),
             pltpu.VMEM((b_per_w, _L), jnp.float32)]
            + [pltpu.VMEM((b_per_w, _L), jnp.float32) for _ in range(_NBUF)]
            + [pltpu.SemaphoreType.DMA for _ in range(_NBUF)]
        ),
    )
    def k(proj_hbm, ids_hbm, out_hbm, idx_v, stage_v, *bufsems):
        bufs = bufsems[:_NBUF]
        sems = bufsems[_NBUF:]
        wid = lax.axis_index("s") * _NC + lax.axis_index("c")
        base = wid * b_per_w
        pltpu.sync_copy(ids_hbm.at[:, wid], idx_v)

        def issue(s, buf, sem):
            pltpu.async_copy(proj_hbm.at[idx_v.at[s]], buf, sem)

        def wait(buf, sem):
            pltpu.make_async_copy(proj_hbm.at[idx_v.at[0]], buf, sem).wait()

        def accum4(quad):
            @pl.loop(0, b_per_w, step=4)
            def _(r):
                for u in range(4):
                    acc = stage_v[r + u]
                    for b in quad:
                        acc = acc + b[r + u]
                    stage_v[r + u] = acc

        @pl.loop(0, b_per_w)
        def _(r):
            stage_v[r] = jnp.zeros((_L,), jnp.float32)

        for b in range(_NBUF):
            issue(b, bufs[b], sems[b])

        @pl.loop(0, S, step=_NBUF)
        def _(s):
            for h in range(2):
                o = 4 * h
                for j in range(4):
                    wait(bufs[o + j], sems[o + j])
                accum4(bufs[o:o + 4])
                for j in range(4):
                    @pl.when(s + _NBUF + o + j < S)
                    def _():
                        issue(s + _NBUF + o + j, bufs[o + j], sems[o + j])

        pltpu.sync_copy(stage_v, out_hbm.at[pl.ds(base, b_per_w)])

    return k(proj, ids_w)


def _finish(sums, b1, W2, b2, S):
    B = sums.shape[0]

    def body(s_ref, b1_ref, w2_ref, b2_ref, o_ref):
        bc = jnp.dot(b1_ref[...], w2_ref[...],
                     preferred_element_type=jnp.float32,
                     precision=lax.Precision.HIGHEST) + b2_ref[...]
        z = s_ref[...][:, :2] * (1.0 / S) + bc
        m = jnp.max(z, axis=1, keepdims=True)
        e = jnp.exp(z - m)
        o_ref[...] = e / jnp.sum(e, axis=1, keepdims=True)

    return pl.pallas_call(
        body,
        out_shape=jax.ShapeDtypeStruct((B, 2), jnp.float32),
    )(sums, b1.reshape(1, -1), W2, b2.reshape(1, -1))


def kernel(input_ids, table, W1, b1, W2, b2):
    B, S = input_ids.shape
    b_per_w = B // _NW
    proj = _project(table, W1, W2).reshape(-1, _L)
    ids_w = _permute_ids(input_ids.astype(jnp.int32)).T.reshape(S, _NW, b_per_w)
    sums = _sc_pool(proj, ids_w, B, S)
    return _finish(sums, b1, W2, b2, S)
